# Initial kernel scaffold; baseline (speedup 1.0000x reference)
import functools

import jax
import jax.numpy as jnp
from jax import lax
from jax.experimental import pallas as pl
from jax.experimental.pallas import tpu as pltpu
from jax.experimental.pallas import tpu_sc as plsc

NC = 2
NS = 16
NW = NC * NS
LANES = 16

B = 4
NVERT = 12306
NFACE = 24576
FEAT = 128
H = 256
W = 256
HW = H * W

NPAD = 12312
CHUNK = 2048
NCHUNK = HW // CHUNK
DGRP = 8
PASSES = (B * FEAT) // (NW * DGRP)


def _index_body(face_hbm, pix_hbm, bary_hbm, comb_hbm, face_v, pix_v, bary_v, comb_v):
    wid = lax.axis_index("s") * NC + lax.axis_index("c")
    pltpu.sync_copy(face_hbm, face_v)
    pltpu.sync_copy(pix_hbm.at[pl.ds(wid * CHUNK, CHUNK)], pix_v)
    pltpu.sync_copy(bary_hbm.at[wid], bary_v)

    def grp(i, carry):
        s = pl.multiple_of(i * LANES, LANES)
        p = pix_v[pl.ds(s, LANES)]
        valid = p >= 0
        pm = jnp.maximum(p, 0)
        i3 = pm * 3
        one = jnp.full((LANES,), 1.0, jnp.float32)
        zero = jnp.full((LANES,), 0.0, jnp.float32)
        mskf = jnp.where(valid, one, zero)
        for k in range(3):
            g = plsc.load_gather(face_v, [i3 + k])
            comb_v[k, pl.ds(s, LANES)] = g
            bk = bary_v[k, pl.ds(s, LANES)] * mskf
            comb_v[3 + k, pl.ds(s, LANES)] = plsc.bitcast(bk, jnp.int32)
        return carry

    lax.fori_loop(0, CHUNK // LANES, grp, 0)
    pltpu.sync_copy(comb_v, comb_hbm.at[wid])


def _gather_body(vt_hbm, comb_hbm, out_hbm, tables_v, comb_v, out_v):
    wid = lax.axis_index("s") * NC + lax.axis_index("c")
    dspl = [jnp.full((LANES,), d, jnp.int32) for d in range(DGRP)]
    for pass_ in range(PASSES):
        q0 = wid * (DGRP * PASSES) + pass_ * DGRP
        pltpu.sync_copy(vt_hbm.at[pl.ds(q0, DGRP)], tables_v)

        def chunk(j, carry):
            c0 = pl.multiple_of(j * CHUNK, CHUNK)
            pltpu.sync_copy(comb_hbm.at[j], comb_v)

            def grp(i, carry2):
                s = pl.multiple_of(i * LANES, LANES)
                idx0 = comb_v[0, pl.ds(s, LANES)]
                idx1 = comb_v[1, pl.ds(s, LANES)]
                idx2 = comb_v[2, pl.ds(s, LANES)]
                b0 = plsc.bitcast(comb_v[3, pl.ds(s, LANES)], jnp.float32)
                b1 = plsc.bitcast(comb_v[4, pl.ds(s, LANES)], jnp.float32)
                b2 = plsc.bitcast(comb_v[5, pl.ds(s, LANES)], jnp.float32)
                for d in range(DGRP):
                    g0 = plsc.load_gather(tables_v, [dspl[d], idx0])
                    g1 = plsc.load_gather(tables_v, [dspl[d], idx1])
                    g2 = plsc.load_gather(tables_v, [dspl[d], idx2])
                    out_v[d, pl.ds(s, LANES)] = b0 * g0 + b1 * g1 + b2 * g2
                return carry2

            lax.fori_loop(0, CHUNK // LANES, grp, 0)
            pltpu.sync_copy(out_v, out_hbm.at[pl.ds(q0, DGRP), pl.ds(c0, CHUNK)])
            return carry

        lax.fori_loop(0, NCHUNK, chunk, 0)


def _mesh():
    return plsc.VectorSubcoreMesh(
        core_axis_name="c", subcore_axis_name="s", num_cores=NC, num_subcores=NS
    )


@functools.partial(
    pl.kernel,
    mesh=_mesh(),
    out_type=jax.ShapeDtypeStruct((NCHUNK, 6, CHUNK), jnp.int32),
    scratch_types=[
        pltpu.VMEM((NFACE * 3,), jnp.int32),
        pltpu.VMEM((CHUNK,), jnp.int32),
        pltpu.VMEM((3, CHUNK), jnp.float32),
        pltpu.VMEM((6, CHUNK), jnp.int32),
    ],
)
def _index_kernel(face_hbm, pix_hbm, bary_hbm, comb_hbm, *scratch):
    _index_body(face_hbm, pix_hbm, bary_hbm, comb_hbm, *scratch)


@functools.partial(
    pl.kernel,
    mesh=_mesh(),
    out_type=jax.ShapeDtypeStruct((B * FEAT, HW), jnp.float32),
    scratch_types=[
        pltpu.VMEM((DGRP, NPAD), jnp.float32),
        pltpu.VMEM((6, CHUNK), jnp.int32),
        pltpu.VMEM((DGRP, CHUNK), jnp.float32),
    ],
)
def _gather_kernel(vt_hbm, comb_hbm, out_hbm, *scratch):
    _gather_body(vt_hbm, comb_hbm, out_hbm, *scratch)


def kernel(vert_feat, bary_coords_uv, pix_to_face_uv, face):
    pix = pix_to_face_uv.reshape(HW).astype(jnp.int32)
    face_flat = face.reshape(NFACE * 3).astype(jnp.int32)
    bary_chunks = bary_coords_uv.reshape(NCHUNK, CHUNK, 3).transpose(0, 2, 1)
    bary_chunks = bary_chunks.astype(jnp.float32)
    vt = jnp.transpose(vert_feat, (0, 2, 1)).reshape(B * FEAT, NVERT)
    vt = jnp.pad(vt, ((0, 0), (0, NPAD - NVERT)))
    comb = _index_kernel(face_flat, pix, bary_chunks)
    out = _gather_kernel(vt, comb)
    return out.reshape(B, FEAT, H, W)


# speedup vs baseline: 2.9049x; 2.9049x over previous
import functools

import jax
import jax.numpy as jnp
from jax import lax
from jax.experimental import pallas as pl
from jax.experimental.pallas import tpu as pltpu
from jax.experimental.pallas import tpu_sc as plsc

NC = 2
NS = 16
NW = NC * NS
LANES = 16

B = 4
NVERT = 12306
NFACE = 24576
FEAT = 128
H = 256
W = 256
HW = H * W

NPAD = 12312
CHUNK = 1024
NCHUNK = HW // CHUNK
CPW = NCHUNK // NW
DGRP = 8
PASSES = (B * FEAT) // (NW * DGRP)


def _index_body(face_hbm, pix_hbm, bary_hbm, comb_hbm, face_v, pix_v, bary_v, comb_v):
    wid = lax.axis_index("s") * NC + lax.axis_index("c")
    pltpu.sync_copy(face_hbm, face_v)
    for m in range(CPW):
        cid = wid * CPW + m
        pltpu.sync_copy(pix_hbm.at[pl.ds(cid * CHUNK, CHUNK)], pix_v)
        pltpu.sync_copy(bary_hbm.at[cid], bary_v)

        def grp(i, carry):
            s = pl.multiple_of(i * LANES, LANES)
            p = pix_v[pl.ds(s, LANES)]
            valid = p >= 0
            pm = jnp.maximum(p, 0)
            i3 = pm * 3
            one = jnp.full((LANES,), 1.0, jnp.float32)
            zero = jnp.full((LANES,), 0.0, jnp.float32)
            mskf = jnp.where(valid, one, zero)
            for k in range(3):
                g = plsc.load_gather(face_v, [i3 + k])
                comb_v[k, pl.ds(s, LANES)] = g
                bk = bary_v[k, pl.ds(s, LANES)] * mskf
                comb_v[3 + k, pl.ds(s, LANES)] = plsc.bitcast(bk, jnp.int32)
            return carry

        lax.fori_loop(0, CHUNK // LANES, grp, 0)
        pltpu.sync_copy(comb_v, comb_hbm.at[cid])


def _gather_body(vt_hbm, comb_hbm, out_hbm, tables_v, comb_v, out_v):
    wid = lax.axis_index("s") * NC + lax.axis_index("c")
    dspl = [jnp.full((LANES,), d, jnp.int32) for d in range(DGRP)]
    for pass_ in range(PASSES):
        q0 = wid * (DGRP * PASSES) + pass_ * DGRP
        pltpu.sync_copy(vt_hbm.at[pl.ds(q0, DGRP)], tables_v)

        def chunk(j, carry):
            c0 = pl.multiple_of(j * CHUNK, CHUNK)
            pltpu.sync_copy(comb_hbm.at[j], comb_v)

            def grp(i, carry2):
                s = pl.multiple_of(i * LANES, LANES)
                idx0 = comb_v[0, pl.ds(s, LANES)]
                idx1 = comb_v[1, pl.ds(s, LANES)]
                idx2 = comb_v[2, pl.ds(s, LANES)]
                b0 = plsc.bitcast(comb_v[3, pl.ds(s, LANES)], jnp.float32)
                b1 = plsc.bitcast(comb_v[4, pl.ds(s, LANES)], jnp.float32)
                b2 = plsc.bitcast(comb_v[5, pl.ds(s, LANES)], jnp.float32)
                for d in range(DGRP):
                    g0 = plsc.load_gather(tables_v, [dspl[d], idx0])
                    g1 = plsc.load_gather(tables_v, [dspl[d], idx1])
                    g2 = plsc.load_gather(tables_v, [dspl[d], idx2])
                    out_v[d, pl.ds(s, LANES)] = b0 * g0 + b1 * g1 + b2 * g2
                return carry2

            lax.fori_loop(0, CHUNK // LANES, grp, 0)
            pltpu.sync_copy(out_v, out_hbm.at[pl.ds(q0, DGRP), pl.ds(c0, CHUNK)])
            return carry

        lax.fori_loop(0, NCHUNK, chunk, 0)


def _mesh():
    return plsc.VectorSubcoreMesh(
        core_axis_name="c", subcore_axis_name="s", num_cores=NC, num_subcores=NS
    )


_PARAMS = pltpu.CompilerParams(needs_layout_passes=False)


@functools.partial(
    pl.kernel,
    mesh=_mesh(),
    compiler_params=_PARAMS,
    out_type=jax.ShapeDtypeStruct((NCHUNK, 6, CHUNK), jnp.int32),
    scratch_types=[
        pltpu.VMEM((NFACE * 3,), jnp.int32),
        pltpu.VMEM((CHUNK,), jnp.int32),
        pltpu.VMEM((3, CHUNK), jnp.float32),
        pltpu.VMEM((6, CHUNK), jnp.int32),
    ],
)
def _index_kernel(face_hbm, pix_hbm, bary_hbm, comb_hbm, *scratch):
    _index_body(face_hbm, pix_hbm, bary_hbm, comb_hbm, *scratch)


@functools.partial(
    pl.kernel,
    mesh=_mesh(),
    compiler_params=_PARAMS,
    out_type=jax.ShapeDtypeStruct((B * FEAT, HW), jnp.float32),
    scratch_types=[
        pltpu.VMEM((DGRP, NPAD), jnp.float32),
        pltpu.VMEM((6, CHUNK), jnp.int32),
        pltpu.VMEM((DGRP, CHUNK), jnp.float32),
    ],
)
def _gather_kernel(vt_hbm, comb_hbm, out_hbm, *scratch):
    _gather_body(vt_hbm, comb_hbm, out_hbm, *scratch)


def kernel(vert_feat, bary_coords_uv, pix_to_face_uv, face):
    pix = pix_to_face_uv.reshape(HW).astype(jnp.int32)
    face_flat = face.reshape(NFACE * 3).astype(jnp.int32)
    bary_chunks = bary_coords_uv.reshape(NCHUNK, CHUNK, 3).transpose(0, 2, 1)
    bary_chunks = bary_chunks.astype(jnp.float32)
    vt = jnp.transpose(vert_feat, (0, 2, 1)).reshape(B * FEAT, NVERT)
    vt = jnp.pad(vt, ((0, 0), (0, NPAD - NVERT)))
    comb = _index_kernel(face_flat, pix, bary_chunks)
    out = _gather_kernel(vt, comb)
    return out.reshape(B, FEAT, H, W)


# speedup vs baseline: 6.5075x; 2.2402x over previous
import functools

import jax
import jax.numpy as jnp
from jax import lax
from jax.experimental import pallas as pl
from jax.experimental.pallas import tpu as pltpu
from jax.experimental.pallas import tpu_sc as plsc

NC = 2
NS = 16
NW = NC * NS
LANES = 16

B = 4
NVERT = 12306
NFACE = 24576
FEAT = 128
H = 256
W = 256
HW = H * W

NPAD = 12312
CHUNK = 512
NCHUNK = HW // CHUNK
CPW = NCHUNK // NW
DGRP = 8
PASSES = (B * FEAT) // (NW * DGRP)
GRPS = CHUNK // LANES


def _index_body(face_hbm, pix_hbm, bary_hbm, comb_hbm, face_v, pix_v, bary_v, comb_v):
    wid = lax.axis_index("s") * NC + lax.axis_index("c")
    pltpu.sync_copy(face_hbm, face_v)
    for m in range(CPW):
        cid = wid * CPW + m
        pltpu.sync_copy(pix_hbm.at[pl.ds(cid * CHUNK, CHUNK)], pix_v)
        pltpu.sync_copy(bary_hbm.at[cid], bary_v)

        @plsc.parallel_loop(0, GRPS)
        def grp(i):
            s = pl.multiple_of(i * LANES, LANES)
            p = pix_v[pl.ds(s, LANES)]
            valid = p >= 0
            pm = jnp.maximum(p, 0)
            i3 = pm * 3
            one = jnp.full((LANES,), 1.0, jnp.float32)
            zero = jnp.full((LANES,), 0.0, jnp.float32)
            mskf = jnp.where(valid, one, zero)
            for k in range(3):
                g = plsc.load_gather(face_v, [i3 + k])
                comb_v[k, pl.ds(s, LANES)] = g
                bk = bary_v[k, pl.ds(s, LANES)] * mskf
                comb_v[3 + k, pl.ds(s, LANES)] = plsc.bitcast(bk, jnp.int32)

        pltpu.sync_copy(comb_v, comb_hbm.at[cid])


def _gather_body(vt_hbm, comb_hbm, out_hbm, tables_v, comb_v, out_v,
                 sem_r0, sem_r1, sem_w0, sem_w1):
    wid = lax.axis_index("s") * NC + lax.axis_index("c")
    sem_r = (sem_r0, sem_r1)
    sem_w = (sem_w0, sem_w1)
    dspl = [jnp.full((LANES,), d, jnp.int32) for d in range(DGRP)]

    def read_start(m, buf):
        pltpu.async_copy(comb_hbm.at[m], comb_v.at[buf], sem_r[buf])

    def read_wait(buf):
        pltpu.make_async_copy(comb_hbm.at[0], comb_v.at[buf], sem_r[buf]).wait()

    def write_start(q0, m, buf):
        pltpu.async_copy(
            out_v.at[buf], out_hbm.at[pl.ds(q0, DGRP), pl.ds(m * CHUNK, CHUNK)],
            sem_w[buf])

    def write_wait(q0, buf):
        pltpu.make_async_copy(
            out_v.at[buf], out_hbm.at[pl.ds(q0, DGRP), pl.ds(0, CHUNK)],
            sem_w[buf]).wait()

    for pass_ in range(PASSES):
        q0 = wid * (DGRP * PASSES) + pass_ * DGRP
        pltpu.sync_copy(vt_hbm.at[pl.ds(q0, DGRP)], tables_v)
        read_start(0, 0)

        def two_chunks(jj, carry):
            for b in range(2):
                m = jj * 2 + b
                read_wait(b)

                @pl.when(m + 1 < NCHUNK)
                def _():
                    read_start(m + 1, 1 - b)

                @pl.when(m >= 2)
                def _():
                    write_wait(q0, b)

                @plsc.parallel_loop(0, GRPS)
                def grp(i):
                    s = pl.multiple_of(i * LANES, LANES)
                    idx0 = comb_v[b, 0, pl.ds(s, LANES)]
                    idx1 = comb_v[b, 1, pl.ds(s, LANES)]
                    idx2 = comb_v[b, 2, pl.ds(s, LANES)]
                    b0 = plsc.bitcast(comb_v[b, 3, pl.ds(s, LANES)], jnp.float32)
                    b1 = plsc.bitcast(comb_v[b, 4, pl.ds(s, LANES)], jnp.float32)
                    b2 = plsc.bitcast(comb_v[b, 5, pl.ds(s, LANES)], jnp.float32)
                    for d in range(DGRP):
                        g0 = plsc.load_gather(tables_v, [dspl[d], idx0])
                        g1 = plsc.load_gather(tables_v, [dspl[d], idx1])
                        g2 = plsc.load_gather(tables_v, [dspl[d], idx2])
                        out_v[b, d, pl.ds(s, LANES)] = b0 * g0 + b1 * g1 + b2 * g2

                write_start(q0, m, b)
            return carry

        lax.fori_loop(0, NCHUNK // 2, two_chunks, 0)
        write_wait(q0, 0)
        write_wait(q0, 1)


def _mesh():
    return plsc.VectorSubcoreMesh(
        core_axis_name="c", subcore_axis_name="s", num_cores=NC, num_subcores=NS
    )


_PARAMS = pltpu.CompilerParams(needs_layout_passes=False)


@functools.partial(
    pl.kernel,
    mesh=_mesh(),
    compiler_params=_PARAMS,
    out_type=jax.ShapeDtypeStruct((NCHUNK, 6, CHUNK), jnp.int32),
    scratch_types=[
        pltpu.VMEM((NFACE * 3,), jnp.int32),
        pltpu.VMEM((CHUNK,), jnp.int32),
        pltpu.VMEM((3, CHUNK), jnp.float32),
        pltpu.VMEM((6, CHUNK), jnp.int32),
    ],
)
def _index_kernel(face_hbm, pix_hbm, bary_hbm, comb_hbm, *scratch):
    _index_body(face_hbm, pix_hbm, bary_hbm, comb_hbm, *scratch)


@functools.partial(
    pl.kernel,
    mesh=_mesh(),
    compiler_params=_PARAMS,
    out_type=jax.ShapeDtypeStruct((B * FEAT, HW), jnp.float32),
    scratch_types=[
        pltpu.VMEM((DGRP, NPAD), jnp.float32),
        pltpu.VMEM((2, 6, CHUNK), jnp.int32),
        pltpu.VMEM((2, DGRP, CHUNK), jnp.float32),
        pltpu.SemaphoreType.DMA,
        pltpu.SemaphoreType.DMA,
        pltpu.SemaphoreType.DMA,
        pltpu.SemaphoreType.DMA,
    ],
)
def _gather_kernel(vt_hbm, comb_hbm, out_hbm, *scratch):
    _gather_body(vt_hbm, comb_hbm, out_hbm, *scratch)


def kernel(vert_feat, bary_coords_uv, pix_to_face_uv, face):
    pix = pix_to_face_uv.reshape(HW).astype(jnp.int32)
    face_flat = face.reshape(NFACE * 3).astype(jnp.int32)
    bary_chunks = bary_coords_uv.reshape(NCHUNK, CHUNK, 3).transpose(0, 2, 1)
    bary_chunks = bary_chunks.astype(jnp.float32)
    vt = jnp.transpose(vert_feat, (0, 2, 1)).reshape(B * FEAT, NVERT)
    vt = jnp.pad(vt, ((0, 0), (0, NPAD - NVERT)))
    comb = _index_kernel(face_flat, pix, bary_chunks)
    out = _gather_kernel(vt, comb)
    return out.reshape(B, FEAT, H, W)
